# baseline (speedup 1.0000x reference)
import functools

import jax
import jax.numpy as jnp
from jax import lax
from jax.experimental import pallas as pl
from jax.experimental.pallas import tpu as pltpu
from jax.experimental.pallas import tpu_sc as plsc

EMB = 32
GRP = 4
NW = 32
CHUNK = 64
RBLK = 16384


def _repack_body(*refs):
    in_refs, out_ref = refs[:-1], refs[-1]
    x = jnp.concatenate([q[0] for q in in_refs], axis=0)
    x = jnp.concatenate(
        [x[:, a * RBLK:(a + 1) * RBLK] for a in range(GRP)], axis=0)
    eye = (lax.broadcasted_iota(jnp.int32, (GRP * EMB, GRP * EMB), 0)
           == lax.broadcasted_iota(jnp.int32, (GRP * EMB, GRP * EMB), 1)
           ).astype(jnp.float32)
    out_ref[...] = lax.dot_general(x, eye, (((0,), (0,)), ((), ())),
                                   preferred_element_type=jnp.float32)


def _pack128(tabT):
    n = tabT.shape[1]
    nblk = -(-n // (GRP * RBLK))
    q = nblk * RBLK
    tabT4 = tabT.reshape(GRP, 8, n)
    in_specs = [
        pl.BlockSpec((1, 8, GRP * RBLK), functools.partial(
            lambda cb, i: (cb, 0, i), cb))
        for cb in range(GRP)
    ]
    return pl.pallas_call(
        _repack_body,
        grid=(nblk,),
        in_specs=in_specs,
        out_specs=pl.BlockSpec((RBLK, GRP * EMB), lambda i: (i, 0)),
        out_shape=jax.ShapeDtypeStruct((q, GRP * EMB), jnp.float32),
    )(*([tabT4] * GRP))


def _sc_gather_groups(gidx, tab, tag):
    b = gidx.shape[0]
    w = b // NW
    nch = w // CHUNK
    d = tab.shape[1]
    mesh = plsc.VectorSubcoreMesh(core_axis_name="core", subcore_axis_name="subcore")

    @functools.partial(
        pl.kernel,
        out_type=jax.ShapeDtypeStruct((b, d), jnp.float32),
        mesh=mesh,
        name=f"gather_{tag}",
        scratch_types=[
            pltpu.VMEM((w,), jnp.int32),
            pltpu.VMEM((CHUNK, 128), jnp.float32),
            pltpu.VMEM((CHUNK, 128), jnp.float32),
            pltpu.SemaphoreType.DMA,
            pltpu.SemaphoreType.DMA,
            pltpu.SemaphoreType.DMA,
        ],
    )
    def gather_kernel(tab_hbm, idx_hbm, out_hbm, idx_v, buf0, buf1,
                      sem_i, sem0, sem1):
        wid = lax.axis_index("subcore") * 2 + lax.axis_index("core")
        base = wid * w
        pltpu.async_copy(idx_hbm.at[pl.ds(base, w)], idx_v, sem_i).wait()

        bufs = (buf0, buf1)
        sems = (sem0, sem1)
        cps = [None, None]
        for c in range(nch):
            p = c & 1
            if cps[p] is not None:
                cps[p].wait()
            cps[p] = pltpu.async_copy(
                tab_hbm.at[idx_v.at[pl.ds(c * CHUNK, CHUNK)]], bufs[p], sems[p])
            if c > 0:
                q = 1 - p
                cps[q].wait()
                cps[q] = None
                pltpu.sync_copy(
                    bufs[q], out_hbm.at[pl.ds(base + (c - 1) * CHUNK, CHUNK)])
        p = (nch - 1) & 1
        cps[p].wait()
        pltpu.sync_copy(
            bufs[p], out_hbm.at[pl.ds(base + (nch - 1) * CHUNK, CHUNK)])

    return gather_kernel(tab, gidx)


def _mlp_body(ug_ref, mg_ref, us_ref, ms_ref, w1u_ref, w1m_ref, b1_ref,
              w2_ref, b2_ref, w3_ref, b3_ref, o_ref):
    dn = (((1,), (1,)), ((), ()))
    ug = ug_ref[...]
    mg = mg_ref[...]
    us = us_ref[...]
    ms = ms_ref[...]
    u = ug[:, :EMB]
    m = mg[:, :EMB]
    for a in range(1, GRP):
        selu = (us == a).astype(jnp.float32)[:, None]
        selm = (ms == a).astype(jnp.float32)[:, None]
        u = jnp.where(selu != 0.0, ug[:, a * EMB:(a + 1) * EMB], u)
        m = jnp.where(selm != 0.0, mg[:, a * EMB:(a + 1) * EMB], m)
    x = jnp.concatenate([u, m], axis=1)
    w1 = jnp.concatenate([w1u_ref[...], w1m_ref[...]], axis=1)
    h = lax.dot_general(x, w1, dn, preferred_element_type=jnp.float32)
    h = jnp.maximum(h + b1_ref[...][None, :], 0.0)
    h = lax.dot_general(h, w2_ref[...], dn,
                        preferred_element_type=jnp.float32)
    h = jnp.maximum(h + b2_ref[...][None, :], 0.0)
    o_ref[...] = jnp.sum(h * w3_ref[...][0][None, :], axis=1) + b3_ref[...]


def _tc_mlp(ugrp, mgrp, usub, msub, W1, b1, W2, b2, W3, b3):
    b = ugrp.shape[0]
    bm = 1024
    w1u = W1[:, :EMB]
    w1m = W1[:, EMB:]
    grid = (b // bm,)
    return pl.pallas_call(
        _mlp_body,
        grid=grid,
        in_specs=[
            pl.BlockSpec((bm, 128), lambda i: (i, 0)),
            pl.BlockSpec((bm, 128), lambda i: (i, 0)),
            pl.BlockSpec((bm,), lambda i: (i,)),
            pl.BlockSpec((bm,), lambda i: (i,)),
            pl.BlockSpec(w1u.shape, lambda i: (0, 0)),
            pl.BlockSpec(w1m.shape, lambda i: (0, 0)),
            pl.BlockSpec(b1.shape, lambda i: (0,)),
            pl.BlockSpec(W2.shape, lambda i: (0, 0)),
            pl.BlockSpec(b2.shape, lambda i: (0,)),
            pl.BlockSpec(W3.shape, lambda i: (0, 0)),
            pl.BlockSpec(b3.shape, lambda i: (0,)),
        ],
        out_specs=pl.BlockSpec((bm,), lambda i: (i,)),
        out_shape=jax.ShapeDtypeStruct((b,), jnp.float32),
    )(ugrp, mgrp, usub, msub, w1u, w1m, b1, W2, b2, W3, b3)


def kernel(user_idx, movie_idx, user_table, movie_table, W1, b1, W2, b2, W3, b3):
    uidx = user_idx.astype(jnp.int32)
    midx = movie_idx.astype(jnp.int32)
    m128 = _pack128(movie_table.T)
    m128, utabT = lax.optimization_barrier((m128, user_table.T))
    u128 = _pack128(utabT)
    ug = (uidx >> 16) * RBLK + (uidx & (RBLK - 1))
    mg = (midx >> 16) * RBLK + (midx & (RBLK - 1))
    us = (uidx & 65535) >> 14
    ms = (midx & 65535) >> 14
    mgrp = _sc_gather_groups(mg, m128, "movie")
    ugrp = _sc_gather_groups(ug, u128, "user")
    return _tc_mlp(ugrp, mgrp, us, ms, W1, b1, W2, b2, W3, b3)


# speedup vs baseline: 1.0258x; 1.0258x over previous
import functools

import jax
import jax.numpy as jnp
from jax import lax
from jax.experimental import pallas as pl
from jax.experimental.pallas import tpu as pltpu
from jax.experimental.pallas import tpu_sc as plsc

EMB = 32
GRP = 4
NW = 32
CHUNK = 64
RBLK = 16384


def _repack_body(*refs):
    in_refs, out_ref = refs[:-1], refs[-1]
    x = jnp.concatenate([q[0] for q in in_refs], axis=0)
    x = jnp.concatenate(
        [x[:, a * RBLK:(a + 1) * RBLK] for a in range(GRP)], axis=0)
    eye = (lax.broadcasted_iota(jnp.int32, (GRP * EMB, GRP * EMB), 0)
           == lax.broadcasted_iota(jnp.int32, (GRP * EMB, GRP * EMB), 1)
           ).astype(jnp.float32)
    out_ref[...] = lax.dot_general(x, eye, (((0,), (0,)), ((), ())),
                                   preferred_element_type=jnp.float32)


def _pack128(tabT):
    n = tabT.shape[1]
    nblk = -(-n // (GRP * RBLK))
    q = nblk * RBLK
    tabT4 = tabT.reshape(GRP, 8, n)
    in_specs = [
        pl.BlockSpec((1, 8, GRP * RBLK), functools.partial(
            lambda cb, i: (cb, 0, i), cb))
        for cb in range(GRP)
    ]
    return pl.pallas_call(
        _repack_body,
        grid=(nblk,),
        in_specs=in_specs,
        out_specs=pl.BlockSpec((RBLK, GRP * EMB), lambda i: (i, 0)),
        out_shape=jax.ShapeDtypeStruct((q, GRP * EMB), jnp.float32),
    )(*([tabT4] * GRP))


def _sc_gather_groups(gidx, tab, tag):
    b = gidx.shape[0]
    w = b // NW
    nch = w // CHUNK
    d = tab.shape[1]
    mesh = plsc.VectorSubcoreMesh(core_axis_name="core", subcore_axis_name="subcore")

    @functools.partial(
        pl.kernel,
        out_type=jax.ShapeDtypeStruct((b, d), jnp.float32),
        mesh=mesh,
        name=f"gather_{tag}",
        scratch_types=[
            pltpu.VMEM((w,), jnp.int32),
            pltpu.VMEM((CHUNK, 128), jnp.float32),
            pltpu.VMEM((CHUNK, 128), jnp.float32),
            pltpu.SemaphoreType.DMA,
            pltpu.SemaphoreType.DMA,
            pltpu.SemaphoreType.DMA,
        ],
    )
    def gather_kernel(tab_hbm, idx_hbm, out_hbm, idx_v, buf0, buf1,
                      sem_i, sem0, sem1):
        wid = lax.axis_index("subcore") * 2 + lax.axis_index("core")
        base = wid * w
        pltpu.async_copy(idx_hbm.at[pl.ds(base, w)], idx_v, sem_i).wait()

        bufs = (buf0, buf1)
        sems = (sem0, sem1)
        cps = [None, None]
        for c in range(nch):
            p = c & 1
            if cps[p] is not None:
                cps[p].wait()
            cps[p] = pltpu.async_copy(
                tab_hbm.at[idx_v.at[pl.ds(c * CHUNK, CHUNK)]], bufs[p], sems[p])
            if c > 0:
                q = 1 - p
                cps[q].wait()
                cps[q] = None
                pltpu.sync_copy(
                    bufs[q], out_hbm.at[pl.ds(base + (c - 1) * CHUNK, CHUNK)])
        p = (nch - 1) & 1
        cps[p].wait()
        pltpu.sync_copy(
            bufs[p], out_hbm.at[pl.ds(base + (nch - 1) * CHUNK, CHUNK)])

    return gather_kernel(tab, gidx)


def _mlp_body(ug_ref, mg_ref, us_ref, ms_ref, w1u_ref, w1m_ref, b1_ref,
              w2_ref, b2_ref, w3_ref, b3_ref, o_ref):
    dn = (((1,), (1,)), ((), ()))
    ug = ug_ref[...]
    mg = mg_ref[...]
    us = us_ref[...]
    ms = ms_ref[...]
    u = ug[:, :EMB]
    m = mg[:, :EMB]
    for a in range(1, GRP):
        selu = (us == a).astype(jnp.float32)[:, None]
        selm = (ms == a).astype(jnp.float32)[:, None]
        u = jnp.where(selu != 0.0, ug[:, a * EMB:(a + 1) * EMB], u)
        m = jnp.where(selm != 0.0, mg[:, a * EMB:(a + 1) * EMB], m)
    x = jnp.concatenate([u, m], axis=1)
    w1 = jnp.concatenate([w1u_ref[...], w1m_ref[...]], axis=1)
    h = lax.dot_general(x, w1, dn, preferred_element_type=jnp.float32)
    h = jnp.maximum(h + b1_ref[...][None, :], 0.0)
    h = lax.dot_general(h, w2_ref[...], dn,
                        preferred_element_type=jnp.float32)
    h = jnp.maximum(h + b2_ref[...][None, :], 0.0)
    o_ref[...] = jnp.sum(h * w3_ref[...][0][None, :], axis=1) + b3_ref[...]


def _tc_mlp(ugrp, mgrp, usub, msub, W1, b1, W2, b2, W3, b3):
    b = ugrp.shape[0]
    bm = 4096
    w1u = W1[:, :EMB]
    w1m = W1[:, EMB:]
    grid = (b // bm,)
    return pl.pallas_call(
        _mlp_body,
        grid=grid,
        in_specs=[
            pl.BlockSpec((bm, 128), lambda i: (i, 0)),
            pl.BlockSpec((bm, 128), lambda i: (i, 0)),
            pl.BlockSpec((bm,), lambda i: (i,)),
            pl.BlockSpec((bm,), lambda i: (i,)),
            pl.BlockSpec(w1u.shape, lambda i: (0, 0)),
            pl.BlockSpec(w1m.shape, lambda i: (0, 0)),
            pl.BlockSpec(b1.shape, lambda i: (0,)),
            pl.BlockSpec(W2.shape, lambda i: (0, 0)),
            pl.BlockSpec(b2.shape, lambda i: (0,)),
            pl.BlockSpec(W3.shape, lambda i: (0, 0)),
            pl.BlockSpec(b3.shape, lambda i: (0,)),
        ],
        out_specs=pl.BlockSpec((bm,), lambda i: (i,)),
        out_shape=jax.ShapeDtypeStruct((b,), jnp.float32),
    )(ugrp, mgrp, usub, msub, w1u, w1m, b1, W2, b2, W3, b3)


def kernel(user_idx, movie_idx, user_table, movie_table, W1, b1, W2, b2, W3, b3):
    uidx = user_idx.astype(jnp.int32)
    midx = movie_idx.astype(jnp.int32)
    m128 = _pack128(movie_table.T)
    m128, utabT = lax.optimization_barrier((m128, user_table.T))
    u128 = _pack128(utabT)
    ug = (uidx >> 16) * RBLK + (uidx & (RBLK - 1))
    mg = (midx >> 16) * RBLK + (midx & (RBLK - 1))
    us = (uidx & 65535) >> 14
    ms = (midx & 65535) >> 14
    mgrp = _sc_gather_groups(mg, m128, "movie")
    ugrp = _sc_gather_groups(ug, u128, "user")
    return _tc_mlp(ugrp, mgrp, us, ms, W1, b1, W2, b2, W3, b3)


# speedup vs baseline: 1.1818x; 1.1521x over previous
import functools

import jax
import jax.numpy as jnp
from jax import lax
from jax.experimental import pallas as pl
from jax.experimental.pallas import tpu as pltpu
from jax.experimental.pallas import tpu_sc as plsc

EMB = 32
GRP = 4
NW = 32
CHUNK = 64
RBLK = 16384


def _repack_body(*refs):
    in_refs, out_ref = refs[:-1], refs[-1]
    x = jnp.concatenate([q[0] for q in in_refs], axis=0)
    x = jnp.concatenate(
        [x[:, a * RBLK:(a + 1) * RBLK] for a in range(GRP)], axis=0)
    eye = (lax.broadcasted_iota(jnp.int32, (GRP * EMB, GRP * EMB), 0)
           == lax.broadcasted_iota(jnp.int32, (GRP * EMB, GRP * EMB), 1)
           ).astype(jnp.float32)
    out_ref[...] = lax.dot_general(x, eye, (((0,), (0,)), ((), ())),
                                   preferred_element_type=jnp.float32)


def _pack128(tabT):
    n = tabT.shape[1]
    nblk = -(-n // (GRP * RBLK))
    q = nblk * RBLK
    tabT4 = tabT.reshape(GRP, 8, n)
    in_specs = [
        pl.BlockSpec((1, 8, GRP * RBLK), functools.partial(
            lambda cb, i: (cb, 0, i), cb))
        for cb in range(GRP)
    ]
    return pl.pallas_call(
        _repack_body,
        grid=(nblk,),
        in_specs=in_specs,
        out_specs=pl.BlockSpec((RBLK, GRP * EMB), lambda i: (i, 0)),
        out_shape=jax.ShapeDtypeStruct((q, GRP * EMB), jnp.float32),
    )(*([tabT4] * GRP))


def _sc_gather_groups(gidx, sub, tab, tag):
    b = gidx.shape[0]
    w = b // NW
    nch = w // CHUNK
    mesh = plsc.VectorSubcoreMesh(core_axis_name="core", subcore_axis_name="subcore")

    @functools.partial(
        pl.kernel,
        out_type=jax.ShapeDtypeStruct((b, EMB), jnp.float32),
        mesh=mesh,
        name=f"gather_{tag}",
        scratch_types=[
            pltpu.VMEM((w,), jnp.int32),
            pltpu.VMEM((w,), jnp.int32),
            pltpu.VMEM((CHUNK, 128), jnp.float32),
            pltpu.VMEM((CHUNK, 128), jnp.float32),
            pltpu.VMEM((CHUNK, EMB), jnp.float32),
            pltpu.VMEM((CHUNK, EMB), jnp.float32),
            pltpu.SemaphoreType.DMA,
            pltpu.SemaphoreType.DMA,
            pltpu.SemaphoreType.DMA,
            pltpu.SemaphoreType.DMA,
        ],
    )
    def gather_kernel(tab_hbm, idx_hbm, sub_hbm, out_hbm, idx_v, sub_v,
                      buf0, buf1, row0, row1, sem_i, sem0, sem1, sem_w):
        wid = lax.axis_index("subcore") * 2 + lax.axis_index("core")
        base = wid * w
        cpi = pltpu.async_copy(idx_hbm.at[pl.ds(base, w)], idx_v, sem_i)
        cps_ = pltpu.async_copy(sub_hbm.at[pl.ds(base, w)], sub_v, sem_i)
        cpi.wait()
        cps_.wait()

        bufs = (buf0, buf1)
        rows = (row0, row1)
        sems = (sem0, sem1)
        cps = [None, None]
        wbs = [None, None]

        def select_and_writeback(c, q):
            buf = bufs[q]
            row = rows[q]
            if wbs[q] is not None:
                wbs[q].wait()
                wbs[q] = None

            @pl.loop(0, CHUNK, step=16)
            def _(cc, _c=c, _buf=buf, _row=row):
                svec = sub_v[pl.ds(_c * CHUNK + cc, 16)]
                for j in range(16):
                    a = svec[j] * EMB
                    _row[cc + j, pl.ds(0, 16)] = _buf[cc + j, pl.ds(a, 16)]
                    _row[cc + j, pl.ds(16, 16)] = _buf[cc + j, pl.ds(a + 16, 16)]

            wbs[q] = pltpu.async_copy(
                row, out_hbm.at[pl.ds(base + c * CHUNK, CHUNK)], sem_w)

        for c in range(nch):
            p = c & 1
            if cps[p] is not None:
                cps[p].wait()
            cps[p] = pltpu.async_copy(
                tab_hbm.at[idx_v.at[pl.ds(c * CHUNK, CHUNK)]], bufs[p], sems[p])
            if c > 0:
                q = 1 - p
                cps[q].wait()
                cps[q] = None
                select_and_writeback(c - 1, q)
        p = (nch - 1) & 1
        cps[p].wait()
        select_and_writeback(nch - 1, p)
        for wb in wbs:
            if wb is not None:
                wb.wait()

    return gather_kernel(tab, gidx, sub)


def _mlp_body(u_ref, m_ref, w1u_ref, w1m_ref, b1_ref,
              w2_ref, b2_ref, w3_ref, b3_ref, o_ref):
    dn = (((1,), (1,)), ((), ()))
    x = jnp.concatenate([u_ref[...], m_ref[...]], axis=1)
    w1 = jnp.concatenate([w1u_ref[...], w1m_ref[...]], axis=1)
    h = lax.dot_general(x, w1, dn, preferred_element_type=jnp.float32)
    h = jnp.maximum(h + b1_ref[...][None, :], 0.0)
    h = lax.dot_general(h, w2_ref[...], dn,
                        preferred_element_type=jnp.float32)
    h = jnp.maximum(h + b2_ref[...][None, :], 0.0)
    o_ref[...] = jnp.sum(h * w3_ref[...][0][None, :], axis=1) + b3_ref[...]


def _tc_mlp(u_vec, m_vec, W1, b1, W2, b2, W3, b3):
    b = u_vec.shape[0]
    bm = 4096
    w1u = W1[:, :EMB]
    w1m = W1[:, EMB:]
    grid = (b // bm,)
    return pl.pallas_call(
        _mlp_body,
        grid=grid,
        in_specs=[
            pl.BlockSpec((bm, EMB), lambda i: (i, 0)),
            pl.BlockSpec((bm, EMB), lambda i: (i, 0)),
            pl.BlockSpec(w1u.shape, lambda i: (0, 0)),
            pl.BlockSpec(w1m.shape, lambda i: (0, 0)),
            pl.BlockSpec(b1.shape, lambda i: (0,)),
            pl.BlockSpec(W2.shape, lambda i: (0, 0)),
            pl.BlockSpec(b2.shape, lambda i: (0,)),
            pl.BlockSpec(W3.shape, lambda i: (0, 0)),
            pl.BlockSpec(b3.shape, lambda i: (0,)),
        ],
        out_specs=pl.BlockSpec((bm,), lambda i: (i,)),
        out_shape=jax.ShapeDtypeStruct((b,), jnp.float32),
    )(u_vec, m_vec, w1u, w1m, b1, W2, b2, W3, b3)


def kernel(user_idx, movie_idx, user_table, movie_table, W1, b1, W2, b2, W3, b3):
    uidx = user_idx.astype(jnp.int32)
    midx = movie_idx.astype(jnp.int32)
    m128 = _pack128(movie_table.T)
    m128, utabT = lax.optimization_barrier((m128, user_table.T))
    u128 = _pack128(utabT)
    ug = (uidx >> 16) * RBLK + (uidx & (RBLK - 1))
    mg = (midx >> 16) * RBLK + (midx & (RBLK - 1))
    us = (uidx & 65535) >> 14
    ms = (midx & 65535) >> 14
    m_vec = _sc_gather_groups(mg, ms, m128, "movie")
    u_vec = _sc_gather_groups(ug, us, u128, "user")
    return _tc_mlp(u_vec, m_vec, W1, b1, W2, b2, W3, b3)


# speedup vs baseline: 1.1999x; 1.0153x over previous
import functools

import jax
import jax.numpy as jnp
from jax import lax
from jax.experimental import pallas as pl
from jax.experimental.pallas import tpu as pltpu
from jax.experimental.pallas import tpu_sc as plsc

EMB = 32
GRP = 4
NW = 32
CHUNK = 128
RBLK = 16384


def _repack_body(*refs):
    in_refs, out_ref = refs[:-1], refs[-1]
    x = jnp.concatenate([q[0] for q in in_refs], axis=0)
    x = jnp.concatenate(
        [x[:, a * RBLK:(a + 1) * RBLK] for a in range(GRP)], axis=0)
    eye = (lax.broadcasted_iota(jnp.int32, (GRP * EMB, GRP * EMB), 0)
           == lax.broadcasted_iota(jnp.int32, (GRP * EMB, GRP * EMB), 1)
           ).astype(jnp.float32)
    out_ref[...] = lax.dot_general(x, eye, (((0,), (0,)), ((), ())),
                                   preferred_element_type=jnp.float32)


def _pack128(tabT):
    n = tabT.shape[1]
    nblk = -(-n // (GRP * RBLK))
    q = nblk * RBLK
    tabT4 = tabT.reshape(GRP, 8, n)
    in_specs = [
        pl.BlockSpec((1, 8, GRP * RBLK), functools.partial(
            lambda cb, i: (cb, 0, i), cb))
        for cb in range(GRP)
    ]
    return pl.pallas_call(
        _repack_body,
        grid=(nblk,),
        in_specs=in_specs,
        out_specs=pl.BlockSpec((RBLK, GRP * EMB), lambda i: (i, 0)),
        out_shape=jax.ShapeDtypeStruct((q, GRP * EMB), jnp.float32),
    )(*([tabT4] * GRP))


def _sc_gather_groups(gidx, sub, tab, tag):
    b = gidx.shape[0]
    w = b // NW
    nch = w // CHUNK
    mesh = plsc.VectorSubcoreMesh(core_axis_name="core", subcore_axis_name="subcore")

    @functools.partial(
        pl.kernel,
        out_type=jax.ShapeDtypeStruct((b, EMB), jnp.float32),
        mesh=mesh,
        name=f"gather_{tag}",
        scratch_types=[
            pltpu.VMEM((w,), jnp.int32),
            pltpu.VMEM((w,), jnp.int32),
            pltpu.VMEM((CHUNK, 128), jnp.float32),
            pltpu.VMEM((CHUNK, 128), jnp.float32),
            pltpu.VMEM((CHUNK, EMB), jnp.float32),
            pltpu.VMEM((CHUNK, EMB), jnp.float32),
            pltpu.SemaphoreType.DMA,
            pltpu.SemaphoreType.DMA,
            pltpu.SemaphoreType.DMA,
            pltpu.SemaphoreType.DMA,
        ],
    )
    def gather_kernel(tab_hbm, idx_hbm, sub_hbm, out_hbm, idx_v, sub_v,
                      buf0, buf1, row0, row1, sem_i, sem0, sem1, sem_w):
        wid = lax.axis_index("subcore") * 2 + lax.axis_index("core")
        base = wid * w
        cpi = pltpu.async_copy(idx_hbm.at[pl.ds(base, w)], idx_v, sem_i)
        cps_ = pltpu.async_copy(sub_hbm.at[pl.ds(base, w)], sub_v, sem_i)
        cpi.wait()
        cps_.wait()

        bufs = (buf0, buf1)
        rows = (row0, row1)
        sems = (sem0, sem1)
        cps = [None, None]
        wbs = [None, None]

        def select_and_writeback(c, q):
            buf = bufs[q]
            row = rows[q]
            if wbs[q] is not None:
                wbs[q].wait()
                wbs[q] = None

            @pl.loop(0, CHUNK, step=16)
            def _(cc, _c=c, _buf=buf, _row=row):
                svec = sub_v[pl.ds(_c * CHUNK + cc, 16)]
                for j in range(16):
                    a = svec[j] * EMB
                    _row[cc + j, pl.ds(0, 16)] = _buf[cc + j, pl.ds(a, 16)]
                    _row[cc + j, pl.ds(16, 16)] = _buf[cc + j, pl.ds(a + 16, 16)]

            wbs[q] = pltpu.async_copy(
                row, out_hbm.at[pl.ds(base + c * CHUNK, CHUNK)], sem_w)

        for c in range(nch):
            p = c & 1
            if cps[p] is not None:
                cps[p].wait()
            cps[p] = pltpu.async_copy(
                tab_hbm.at[idx_v.at[pl.ds(c * CHUNK, CHUNK)]], bufs[p], sems[p])
            if c > 0:
                q = 1 - p
                cps[q].wait()
                cps[q] = None
                select_and_writeback(c - 1, q)
        p = (nch - 1) & 1
        cps[p].wait()
        select_and_writeback(nch - 1, p)
        for wb in wbs:
            if wb is not None:
                wb.wait()

    return gather_kernel(tab, gidx, sub)


def _mlp_body(u_ref, m_ref, w1u_ref, w1m_ref, b1_ref,
              w2_ref, b2_ref, w3_ref, b3_ref, o_ref):
    dn = (((1,), (1,)), ((), ()))
    x = jnp.concatenate([u_ref[...], m_ref[...]], axis=1)
    w1 = jnp.concatenate([w1u_ref[...], w1m_ref[...]], axis=1)
    h = lax.dot_general(x, w1, dn, preferred_element_type=jnp.float32)
    h = jnp.maximum(h + b1_ref[...][None, :], 0.0)
    h = lax.dot_general(h, w2_ref[...], dn,
                        preferred_element_type=jnp.float32)
    h = jnp.maximum(h + b2_ref[...][None, :], 0.0)
    o_ref[...] = jnp.sum(h * w3_ref[...][0][None, :], axis=1) + b3_ref[...]


def _tc_mlp(u_vec, m_vec, W1, b1, W2, b2, W3, b3):
    b = u_vec.shape[0]
    bm = 4096
    w1u = W1[:, :EMB]
    w1m = W1[:, EMB:]
    grid = (b // bm,)
    return pl.pallas_call(
        _mlp_body,
        grid=grid,
        in_specs=[
            pl.BlockSpec((bm, EMB), lambda i: (i, 0)),
            pl.BlockSpec((bm, EMB), lambda i: (i, 0)),
            pl.BlockSpec(w1u.shape, lambda i: (0, 0)),
            pl.BlockSpec(w1m.shape, lambda i: (0, 0)),
            pl.BlockSpec(b1.shape, lambda i: (0,)),
            pl.BlockSpec(W2.shape, lambda i: (0, 0)),
            pl.BlockSpec(b2.shape, lambda i: (0,)),
            pl.BlockSpec(W3.shape, lambda i: (0, 0)),
            pl.BlockSpec(b3.shape, lambda i: (0,)),
        ],
        out_specs=pl.BlockSpec((bm,), lambda i: (i,)),
        out_shape=jax.ShapeDtypeStruct((b,), jnp.float32),
    )(u_vec, m_vec, w1u, w1m, b1, W2, b2, W3, b3)


def kernel(user_idx, movie_idx, user_table, movie_table, W1, b1, W2, b2, W3, b3):
    uidx = user_idx.astype(jnp.int32)
    midx = movie_idx.astype(jnp.int32)
    m128 = _pack128(movie_table.T)
    m128, utabT = lax.optimization_barrier((m128, user_table.T))
    u128 = _pack128(utabT)
    ug = (uidx >> 16) * RBLK + (uidx & (RBLK - 1))
    mg = (midx >> 16) * RBLK + (midx & (RBLK - 1))
    us = (uidx & 65535) >> 14
    ms = (midx & 65535) >> 14
    m_vec = _sc_gather_groups(mg, ms, m128, "movie")
    u_vec = _sc_gather_groups(ug, us, u128, "user")
    return _tc_mlp(u_vec, m_vec, W1, b1, W2, b2, W3, b3)
